# baseline (speedup 1.0000x reference)
import jax
import jax.numpy as jnp
from jax import lax
from jax.experimental import pallas as pl
from jax.experimental.pallas import tpu as pltpu
from jax.experimental.pallas import tpu_sc as plsc

_D = 64
_NC = 2
_NS = 16
_NW = _NC * _NS
_BT_PER_W = 4
_S = 200
_NBLK = _S * _BT_PER_W


def _body(xT_hbm, tpad_hbm, out_hbm,
          ib0, ib1, ib2, ib3, r0, r1, t0, t1,
          is0, is1, is2, is3, gs0, gs1, os0, os1):
    wid = lax.axis_index("s") * _NC + lax.axis_index("c")
    ibs = (ib0, ib1, ib2, ib3)
    rows = (r0, r1)
    ts = (t0, t1)
    isems = (is0, is1, is2, is3)
    gsems = (gs0, gs1)
    osems = (os0, os1)
    bt0 = wid * _BT_PER_W
    iota16 = lax.iota(jnp.int32, 16)

    def blk(k):
        return k // _BT_PER_W, bt0 + lax.rem(k, _BT_PER_W)

    def start_idx(k, q):
        s, bt = blk(k)
        pltpu.async_copy(xT_hbm.at[s, pl.ds(bt * 128, 128)], ibs[q], isems[q])

    def wait_idx(q):
        pltpu.make_async_copy(
            xT_hbm.at[0, pl.ds(0, 128)], ibs[q], isems[q]).wait()

    def start_gather(q, p):
        pltpu.async_copy(tpad_hbm.at[ibs[q]], rows[p], gsems[p])

    def wait_gather(p):
        pltpu.make_async_copy(tpad_hbm.at[ibs[0]], rows[p], gsems[p]).wait()

    def start_out(k, p):
        s, bt = blk(k)
        pltpu.async_copy(ts[p], out_hbm.at[s, :, bt], osems[p])

    def wait_out(p):
        pltpu.make_async_copy(ts[p], out_hbm.at[0, :, 0], osems[p]).wait()

    def transpose(p):
        r = rows[p]
        t = ts[p]
        for dt in range(8):
            for ds_ in range(8):
                dvec = jnp.full((16,), dt * 8 + ds_, jnp.int32)
                for g in range(8):
                    v = plsc.load_gather(r, [iota16 + g * 16, dvec])
                    t[dt, ds_, pl.ds(g * 16, 16)] = v

    for q in range(4):
        start_idx(q, q)
    wait_idx(0)
    start_gather(0, 0)
    wait_idx(1)
    start_gather(1, 1)

    def loop(j, carry):
        for i in range(4):
            k = 4 * j + i
            p = i % 2
            wait_gather(p)

            @pl.when(k >= 2)
            def _():
                wait_out(p)

            transpose(p)
            start_out(k, p)

            @pl.when(k + 4 < _NBLK)
            def _():
                start_idx(k + 4, i)

            @pl.when(k + 2 < _NBLK)
            def _():
                wait_idx((i + 2) % 4)
                start_gather((i + 2) % 4, p)
        return carry

    lax.fori_loop(0, _NBLK // 4, loop, 0)
    wait_out(0)
    wait_out(1)


def kernel(x, table):
    b, s = x.shape
    xT = x.T.astype(jnp.int32)
    tpad = jnp.pad(table, ((0, 0), (0, 64)))
    mesh = plsc.VectorSubcoreMesh(core_axis_name="c", subcore_axis_name="s")
    out5 = pl.kernel(
        _body,
        out_type=jax.ShapeDtypeStruct((_S, 8, 128, 8, 128), jnp.float32),
        mesh=mesh,
        scratch_types=(
            [pltpu.VMEM((128,), jnp.int32)] * 4
            + [pltpu.VMEM((128, 128), jnp.float32)] * 2
            + [pltpu.VMEM((8, 8, 128), jnp.float32)] * 2
            + [pltpu.SemaphoreType.DMA] * 8
        ),
        compiler_params=pltpu.CompilerParams(
            use_tc_tiling_on_sc=True, needs_layout_passes=False),
    )(xT, tpad)
    return out5.transpose(2, 4, 0, 1, 3).reshape(b, s, _D)


# speedup vs baseline: 2.8060x; 2.8060x over previous
import jax
import jax.numpy as jnp
from jax import lax
from jax.experimental import pallas as pl
from jax.experimental.pallas import tpu as pltpu
from jax.experimental.pallas import tpu_sc as plsc

_D = 64
_NC = 2
_NS = 16
_NW = _NC * _NS
_BT_PER_W = 4
_S = 200
_NBLK = _S * _BT_PER_W
_SBB = 16


def _body(xT_hbm, tr_hbm, out_hbm,
          ib, ib2, r0, r1, t0, t1,
          isem, gs0, gs1, os0, os1):
    wid = lax.axis_index("s") * _NC + lax.axis_index("c")
    rows = (r0, r1)
    ts = (t0, t1)
    gsems = (gs0, gs1)
    osems = (os0, os1)
    bt0 = wid * _BT_PER_W
    col0 = bt0 * 128
    iota16 = lax.iota(jnp.int32, 16)
    bvs = [iota16 + g * 16 for g in range(8)]

    def stage_idx(m):
        pltpu.async_copy(
            xT_hbm.at[pl.ds(m * 4, 4), pl.ds(col0, 512)],
            ib.at[lax.rem(m, 2)], isem)

    def wait_and_shift(m):
        pltpu.make_async_copy(
            xT_hbm.at[pl.ds(0, 4), pl.ds(0, 512)], ib.at[0], isem).wait()
        mp = lax.rem(m, 2)

        def sh(ls, carry):
            def sh2(c, carry2):
                v = ib[mp, ls, pl.ds(c * 16, 16)]
                ib2[mp, ls, pl.ds(c * 16, 16)] = lax.shift_right_logical(v, 1)
                return carry2
            return lax.fori_loop(0, 32, sh2, carry)
        lax.fori_loop(0, 4, sh, 0)

    def start_gather(k, p):
        mp = lax.rem(k // _SBB, 2)
        ls = lax.rem(k // _BT_PER_W, 4)
        bo = lax.rem(k, _BT_PER_W)
        pltpu.async_copy(
            tr_hbm.at[ib2.at[mp, ls, pl.ds(bo * 128, 128)]],
            rows[p], gsems[p])

    def wait_gather(p):
        pltpu.make_async_copy(
            tr_hbm.at[ib2.at[0, 0, pl.ds(0, 128)]], rows[p], gsems[p]).wait()

    def start_out(k, p):
        s = k // _BT_PER_W
        bt = bt0 + lax.rem(k, _BT_PER_W)
        pltpu.async_copy(ts[p], out_hbm.at[s, :, bt], osems[p])

    def wait_out(p):
        pltpu.make_async_copy(ts[p], out_hbm.at[0, :, 0], osems[p]).wait()

    def transpose(k, p):
        mp = lax.rem(k // _SBB, 2)
        ls = lax.rem(k // _BT_PER_W, 4)
        bo = lax.rem(k, _BT_PER_W)
        r = rows[p]
        t = ts[p]
        pars = []
        for g in range(8):
            iv = ib[mp, ls, pl.ds(bo * 128 + g * 16, 16)]
            pars.append(lax.shift_left(lax.bitwise_and(iv, 1), 6))

        def diag(jd, carry):
            rot_j = lax.rem(iota16 + jd, jnp.full((16,), 16, jnp.int32))
            for dq in range(4):
                idx_d = rot_j + dq * 16
                idx_dt = lax.shift_right_logical(idx_d, 3)
                idx_ds = lax.bitwise_and(idx_d, 7)
                for g in range(8):
                    v = plsc.load_gather(r, [bvs[g], idx_d + pars[g]])
                    plsc.store_scatter(t, [idx_dt, idx_ds, bvs[g]], v)
            return carry

        lax.fori_loop(0, 16, diag, 0)

    stage_idx(0)
    stage_idx(1)
    wait_and_shift(0)
    start_gather(0, 0)

    def loop(j, carry):
        for i in range(2):
            k = 2 * j + i
            p = i
            wait_gather(p)
            if i == 0:
                @pl.when((k > 0) & (lax.rem(k, _SBB) == 0) & (k + _SBB < _NBLK))
                def _():
                    stage_idx(k // _SBB + 1)

                @pl.when(k + 1 < _NBLK)
                def _():
                    start_gather(k + 1, 1)
            else:
                @pl.when(k + 1 < _NBLK)
                def _():
                    @pl.when(lax.rem(k + 1, _SBB) == 0)
                    def _():
                        wait_and_shift((k + 1) // _SBB)
                    start_gather(k + 1, 0)

            @pl.when(k >= 2)
            def _():
                wait_out(p)

            transpose(k, p)
            start_out(k, p)
        return carry

    lax.fori_loop(0, _NBLK // 2, loop, 0)
    wait_out(0)
    wait_out(1)


def kernel(x, table):
    b, s = x.shape
    xT = x.T.astype(jnp.int32)
    tr = table.reshape(table.shape[0] // 2, 128)
    mesh = plsc.VectorSubcoreMesh(core_axis_name="c", subcore_axis_name="s")
    out5 = pl.kernel(
        _body,
        out_type=jax.ShapeDtypeStruct((_S, 8, 128, 8, 128), jnp.float32),
        mesh=mesh,
        scratch_types=(
            [pltpu.VMEM((2, 4, 512), jnp.int32)] * 2
            + [pltpu.VMEM((128, 128), jnp.float32)] * 2
            + [pltpu.VMEM((8, 8, 128), jnp.float32)] * 2
            + [pltpu.SemaphoreType.DMA] * 5
        ),
        compiler_params=pltpu.CompilerParams(
            use_tc_tiling_on_sc=True, needs_layout_passes=False),
    )(xT, tr)
    return out5.transpose(2, 4, 0, 1, 3).reshape(b, s, _D)
